# Initial kernel scaffold; baseline (speedup 1.0000x reference)
import functools

import jax
import jax.numpy as jnp
from jax import lax
from jax.experimental import pallas as pl
from jax.experimental.pallas import tpu as pltpu
from jax.experimental.pallas import tpu_sc as plsc

_N_SUBCORES = 16
_CHUNK = 80


def _tc_transform(ns, w, a2):
    m, d = ns.shape
    units = w.shape[1]
    half = units // 2
    bm = 1000

    def body(ns_ref, w_ref, a_ref, h0_ref, h1_ref, s_ref):
        x = ns_ref[...]
        h = jnp.dot(x, w_ref[...], precision=lax.Precision.HIGHEST)
        h0_ref[...] = h[:, :half]
        h1_ref[...] = h[:, half:]
        s_ref[...] = lax.dot_general(
            h, a_ref[...], (((1,), (1,)), ((), ())),
            precision=lax.Precision.HIGHEST)

    return pl.pallas_call(
        body,
        grid=(m // bm,),
        in_specs=[
            pl.BlockSpec((bm, d), lambda i: (i, 0)),
            pl.BlockSpec((d, units), lambda i: (0, 0)),
            pl.BlockSpec((2, units), lambda i: (0, 0)),
        ],
        out_specs=[
            pl.BlockSpec((bm, half), lambda i: (i, 0)),
            pl.BlockSpec((bm, half), lambda i: (i, 0)),
            pl.BlockSpec((bm, 2), lambda i: (i, 0)),
        ],
        out_shape=[
            jax.ShapeDtypeStruct((m, half), jnp.float32),
            jax.ShapeDtypeStruct((m, half), jnp.float32),
            jax.ShapeDtypeStruct((m, 2), jnp.float32),
        ],
    )(ns, w, a2)


def _sc_gat(s1, s2, h0, h1, dst3, src3, n_nodes, n_pad):
    half = h0.shape[1]
    n_chunks = dst3.shape[1]
    rows_per_tile = n_pad // _N_SUBCORES
    n_sub = rows_per_tile // _CHUNK
    mesh = plsc.VectorSubcoreMesh(core_axis_name="core", subcore_axis_name="subcore")

    @functools.partial(
        pl.kernel,
        out_type=jax.ShapeDtypeStruct((2, n_pad, half), jnp.float32),
        mesh=mesh,
        scratch_types=[
            pltpu.VMEM((n_nodes,), jnp.float32),
            pltpu.VMEM((n_nodes,), jnp.float32),
            pltpu.VMEM((n_chunks, _CHUNK), jnp.int32),
            pltpu.VMEM((n_chunks, _CHUNK), jnp.int32),
            pltpu.VMEM((n_chunks, _CHUNK), jnp.float32),
            pltpu.VMEM((_CHUNK, half), jnp.float32),
            pltpu.VMEM((n_pad // _N_SUBCORES,), jnp.float32),
            pltpu.VMEM((n_pad // _N_SUBCORES,), jnp.float32),
            pltpu.VMEM_SHARED((n_pad,), jnp.float32),
            pltpu.VMEM_SHARED((n_pad, half), jnp.float32),
        ],
    )
    def k(s1_hbm, s2_hbm, h0_hbm, h1_hbm, dst_hbm, src_hbm, out_hbm,
          s1_v, s2_v, dstv, srcv, e_v, rows, dn_v, rec_v, denom_sp, out_sp):
        core = lax.axis_index("core")
        t = lax.axis_index("subcore")
        base = t * rows_per_tile

        pltpu.sync_copy(s1_hbm, s1_v)
        pltpu.sync_copy(s2_hbm, s2_v)
        pltpu.sync_copy(dst_hbm.at[t], dstv)
        pltpu.sync_copy(src_hbm.at[t], srcv)

        zeros16 = jnp.zeros((16,), jnp.float32)

        @pl.loop(0, _CHUNK)
        def _(j):
            for q in range(half // 16):
                rows[j, pl.ds(q * 16, 16)] = zeros16

        for ksub in range(n_sub):
            pltpu.sync_copy(rows, out_sp.at[pl.ds(base + ksub * _CHUNK, _CHUNK)])

        @pl.loop(0, rows_per_tile // 16)
        def _(g):
            dn_v[pl.ds(g * 16, 16)] = zeros16

        pltpu.sync_copy(dn_v, denom_sp.at[pl.ds(base, rows_per_tile)])

        plsc.subcore_barrier()

        @pl.loop(0, n_chunks)
        def _(c):
            for g in range(_CHUNK // 16):
                d16 = dstv[c, pl.ds(g * 16, 16)]
                sx16 = srcv[c, pl.ds(g * 16, 16)]
                a = plsc.load_gather(s1_v, [d16])
                b = plsc.load_gather(s2_v, [sx16])
                x = a + b
                x = jnp.maximum(x, x * 0.2)
                x = jnp.minimum(jnp.maximum(x, -2.0), 2.0)
                e_v[c, pl.ds(g * 16, 16)] = jnp.exp(x)
            pltpu.sync_copy(e_v.at[c], denom_sp.at[dstv.at[c]], add=True)

        def p2(h_ref):
            @pl.loop(0, n_chunks)
            def _(c):
                pltpu.sync_copy(h_ref.at[srcv.at[c]], rows)

                @pl.loop(0, _CHUNK)
                def _(j):
                    ej = e_v[c, j]
                    for q in range(half // 16):
                        rows[j, pl.ds(q * 16, 16)] = rows[j, pl.ds(q * 16, 16)] * ej

                pltpu.sync_copy(rows, out_sp.at[dstv.at[c]], add=True)

        @pl.when(core == 0)
        def _():
            p2(h0_hbm)

        @pl.when(core == 1)
        def _():
            p2(h1_hbm)

        plsc.subcore_barrier()

        pltpu.sync_copy(denom_sp.at[pl.ds(base, rows_per_tile)], dn_v)

        @pl.loop(0, rows_per_tile // 16)
        def _(g):
            d16 = dn_v[pl.ds(g * 16, 16)]
            rec_v[pl.ds(g * 16, 16)] = 1.0 / jnp.maximum(d16, 1e-20)

        for ksub in range(n_sub):
            off = base + ksub * _CHUNK
            pltpu.sync_copy(out_sp.at[pl.ds(off, _CHUNK)], rows)

            @pl.loop(0, _CHUNK)
            def _(j):
                r = rec_v[ksub * _CHUNK + j]
                for q in range(half // 16):
                    rows[j, pl.ds(q * 16, 16)] = rows[j, pl.ds(q * 16, 16)] * r

            pltpu.sync_copy(rows, out_hbm.at[core, pl.ds(off, _CHUNK)])

    return k(s1, s2, h0, h1, dst3, src3)


def kernel(node_states, edges, kernel, kernel_attention):
    n_nodes, _ = node_states.shape
    n_edges = edges.shape[0]
    n_pad = ((n_nodes + 16 * _CHUNK - 1) // (16 * _CHUNK)) * (16 * _CHUNK)
    per_tile = n_edges // _N_SUBCORES
    n_chunks = per_tile // _CHUNK

    dst3 = edges[:, 0].reshape(_N_SUBCORES, n_chunks, _CHUNK)
    src3 = edges[:, 1].reshape(_N_SUBCORES, n_chunks, _CHUNK)
    a2 = kernel_attention.reshape(2, kernel.shape[1])

    h0, h1, s = _tc_transform(node_states, kernel, a2)
    out2 = _sc_gat(s[:, 0], s[:, 1], h0, h1, dst3, src3, n_nodes, n_pad)
    return jnp.concatenate([out2[0, :n_nodes], out2[1, :n_nodes]], axis=1)


# speedup vs baseline: 16.1786x; 16.1786x over previous
import dataclasses
import functools

import jax
import jax.numpy as jnp
from jax import lax
from jax.experimental import pallas as pl
from jax.experimental.pallas import tpu as pltpu
from jax.experimental.pallas import tpu_sc as plsc

_N_SUBCORES = 16
_CHUNK = 80


def _tc_transform(ns, w, a2):
    m, d = ns.shape
    units = w.shape[1]
    half = units // 2
    bm = 1000

    def body(ns_ref, w_ref, a_ref, h0_ref, h1_ref, s_ref):
        x = ns_ref[...]
        h = jnp.dot(x, w_ref[...], precision=lax.Precision.HIGHEST)
        h0_ref[...] = h[:, :half]
        h1_ref[...] = h[:, half:]
        s_ref[...] = lax.dot_general(
            h, a_ref[...], (((1,), (1,)), ((), ())),
            precision=lax.Precision.HIGHEST)

    return pl.pallas_call(
        body,
        grid=(m // bm,),
        in_specs=[
            pl.BlockSpec((bm, d), lambda i: (i, 0)),
            pl.BlockSpec((d, units), lambda i: (0, 0)),
            pl.BlockSpec((2, units), lambda i: (0, 0)),
        ],
        out_specs=[
            pl.BlockSpec((bm, half), lambda i: (i, 0)),
            pl.BlockSpec((bm, half), lambda i: (i, 0)),
            pl.BlockSpec((bm, 2), lambda i: (i, 0)),
        ],
        out_shape=[
            jax.ShapeDtypeStruct((m, half), jnp.float32),
            jax.ShapeDtypeStruct((m, half), jnp.float32),
            jax.ShapeDtypeStruct((m, 2), jnp.float32),
        ],
    )(ns, w, a2)


def _sc_gat(s1, s2, h0, h1, dst3, src3, n_nodes, n_pad):
    half = h0.shape[1]
    n_chunks = dst3.shape[1]
    rows_per_tile = n_pad // _N_SUBCORES
    n_sub = rows_per_tile // _CHUNK
    mesh = plsc.VectorSubcoreMesh(core_axis_name="core", subcore_axis_name="subcore")
    cp = pltpu.CompilerParams()
    if "needs_layout_passes" in pltpu.CompilerParams.__dataclass_fields__:
        cp = dataclasses.replace(cp, needs_layout_passes=False)
    if "use_tc_tiling_on_sc" in pltpu.CompilerParams.__dataclass_fields__:
        cp = dataclasses.replace(cp, use_tc_tiling_on_sc=False)

    @functools.partial(
        pl.kernel,
        out_type=jax.ShapeDtypeStruct((2, n_pad, half), jnp.float32),
        mesh=mesh,
        compiler_params=cp,
        scratch_types=[
            pltpu.VMEM((n_nodes,), jnp.float32),
            pltpu.VMEM((n_nodes,), jnp.float32),
            pltpu.VMEM((n_chunks, _CHUNK), jnp.int32),
            pltpu.VMEM((n_chunks, _CHUNK), jnp.int32),
            pltpu.VMEM((n_chunks, _CHUNK), jnp.float32),
            pltpu.VMEM((_CHUNK, half), jnp.float32),
            pltpu.VMEM((n_pad // _N_SUBCORES,), jnp.float32),
            pltpu.VMEM((n_pad // _N_SUBCORES,), jnp.float32),
            pltpu.VMEM_SHARED((n_pad,), jnp.float32),
            pltpu.VMEM_SHARED((n_pad, half), jnp.float32),
        ],
    )
    def k(s1_hbm, s2_hbm, h0_hbm, h1_hbm, dst_hbm, src_hbm, out_hbm,
          s1_v, s2_v, dstv, srcv, e_v, rows, dn_v, rec_v, denom_sp, out_sp):
        core = lax.axis_index("core")
        t = lax.axis_index("subcore")
        base = t * rows_per_tile

        pltpu.sync_copy(s1_hbm, s1_v)
        pltpu.sync_copy(s2_hbm, s2_v)
        pltpu.sync_copy(dst_hbm.at[t], dstv)
        pltpu.sync_copy(src_hbm.at[t], srcv)

        zeros16 = jnp.zeros((16,), jnp.float32)

        @pl.loop(0, _CHUNK)
        def _(j):
            for q in range(half // 16):
                rows[j, pl.ds(q * 16, 16)] = zeros16

        for ksub in range(n_sub):
            pltpu.sync_copy(rows, out_sp.at[pl.ds(base + ksub * _CHUNK, _CHUNK)])

        @pl.loop(0, rows_per_tile // 16)
        def _(g):
            dn_v[pl.ds(g * 16, 16)] = zeros16

        pltpu.sync_copy(dn_v, denom_sp.at[pl.ds(base, rows_per_tile)])

        plsc.subcore_barrier()

        @pl.loop(0, n_chunks)
        def _(c):
            for g in range(_CHUNK // 16):
                d16 = dstv[c, pl.ds(g * 16, 16)]
                sx16 = srcv[c, pl.ds(g * 16, 16)]
                a = plsc.load_gather(s1_v, [d16])
                b = plsc.load_gather(s2_v, [sx16])
                x = a + b
                x = jnp.maximum(x, x * 0.2)
                x = jnp.minimum(jnp.maximum(x, -2.0), 2.0)
                e_v[c, pl.ds(g * 16, 16)] = jnp.exp(x)
            pltpu.sync_copy(e_v.at[c], denom_sp.at[dstv.at[c]], add=True)

        def p2(h_ref):
            @pl.loop(0, n_chunks)
            def _(c):
                pltpu.sync_copy(h_ref.at[srcv.at[c]], rows)

                for g in range(_CHUNK // 16):
                    e16 = e_v[c, pl.ds(g * 16, 16)]
                    for jj in range(16):
                        j = g * 16 + jj
                        ej = e16[jj]
                        for q in range(half // 16):
                            rows[j, pl.ds(q * 16, 16)] = rows[j, pl.ds(q * 16, 16)] * ej

                pltpu.sync_copy(rows, out_sp.at[dstv.at[c]], add=True)

        @pl.when(core == 0)
        def _():
            p2(h0_hbm)

        @pl.when(core == 1)
        def _():
            p2(h1_hbm)

        plsc.subcore_barrier()

        pltpu.sync_copy(denom_sp.at[pl.ds(base, rows_per_tile)], dn_v)

        @pl.loop(0, rows_per_tile // 16)
        def _(g):
            d16 = dn_v[pl.ds(g * 16, 16)]
            rec_v[pl.ds(g * 16, 16)] = 1.0 / jnp.maximum(d16, 1e-20)

        for ksub in range(n_sub):
            off = base + ksub * _CHUNK
            pltpu.sync_copy(out_sp.at[pl.ds(off, _CHUNK)], rows)

            for g in range(_CHUNK // 16):
                r16 = rec_v[pl.ds(ksub * _CHUNK + g * 16, 16)]
                for jj in range(16):
                    j = g * 16 + jj
                    rj = r16[jj]
                    for q in range(half // 16):
                        rows[j, pl.ds(q * 16, 16)] = rows[j, pl.ds(q * 16, 16)] * rj

            pltpu.sync_copy(rows, out_hbm.at[core, pl.ds(off, _CHUNK)])

    return k(s1, s2, h0, h1, dst3, src3)


def kernel(node_states, edges, kernel, kernel_attention):
    n_nodes, _ = node_states.shape
    n_edges = edges.shape[0]
    n_pad = ((n_nodes + 16 * _CHUNK - 1) // (16 * _CHUNK)) * (16 * _CHUNK)
    per_tile = n_edges // _N_SUBCORES
    n_chunks = per_tile // _CHUNK

    dst3 = edges[:, 0].reshape(_N_SUBCORES, n_chunks, _CHUNK)
    src3 = edges[:, 1].reshape(_N_SUBCORES, n_chunks, _CHUNK)
    a2 = kernel_attention.reshape(2, kernel.shape[1])

    h0, h1, s = _tc_transform(node_states, kernel, a2)
    out2 = _sc_gat(s[:, 0], s[:, 1], h0, h1, dst3, src3, n_nodes, n_pad)
    return jnp.concatenate([out2[0, :n_nodes], out2[1, :n_nodes]], axis=1)


# speedup vs baseline: 27.6752x; 1.7106x over previous
import dataclasses
import functools

import jax
import jax.numpy as jnp
from jax import lax
from jax.experimental import pallas as pl
from jax.experimental.pallas import tpu as pltpu
from jax.experimental.pallas import tpu_sc as plsc

_N_SUBCORES = 16
_CHUNK = 80


def _tc_transform(ns, w, a2):
    m, d = ns.shape
    units = w.shape[1]
    half = units // 2
    bm = 1000

    def body(ns_ref, w_ref, a_ref, h0_ref, h1_ref, s_ref):
        x = ns_ref[...]
        h = jnp.dot(x, w_ref[...], precision=lax.Precision.HIGHEST)
        h0_ref[...] = h[:, :half]
        h1_ref[...] = h[:, half:]
        s_ref[...] = lax.dot_general(
            h, a_ref[...], (((1,), (1,)), ((), ())),
            precision=lax.Precision.HIGHEST)

    return pl.pallas_call(
        body,
        grid=(m // bm,),
        in_specs=[
            pl.BlockSpec((bm, d), lambda i: (i, 0)),
            pl.BlockSpec((d, units), lambda i: (0, 0)),
            pl.BlockSpec((2, units), lambda i: (0, 0)),
        ],
        out_specs=[
            pl.BlockSpec((bm, half), lambda i: (i, 0)),
            pl.BlockSpec((bm, half), lambda i: (i, 0)),
            pl.BlockSpec((bm, 2), lambda i: (i, 0)),
        ],
        out_shape=[
            jax.ShapeDtypeStruct((m, half), jnp.float32),
            jax.ShapeDtypeStruct((m, half), jnp.float32),
            jax.ShapeDtypeStruct((m, 2), jnp.float32),
        ],
    )(ns, w, a2)


def _sc_gat(s1, s2, h0, h1, dst3, src3, n_nodes, n_pad):
    half = h0.shape[1]
    n_chunks = dst3.shape[1]
    rows_per_tile = n_pad // _N_SUBCORES
    n_sub = rows_per_tile // _CHUNK
    mesh = plsc.VectorSubcoreMesh(core_axis_name="core", subcore_axis_name="subcore")
    cp = pltpu.CompilerParams()
    if "needs_layout_passes" in pltpu.CompilerParams.__dataclass_fields__:
        cp = dataclasses.replace(cp, needs_layout_passes=False)
    if "use_tc_tiling_on_sc" in pltpu.CompilerParams.__dataclass_fields__:
        cp = dataclasses.replace(cp, use_tc_tiling_on_sc=False)

    @functools.partial(
        pl.kernel,
        out_type=jax.ShapeDtypeStruct((2, n_pad, half), jnp.float32),
        mesh=mesh,
        compiler_params=cp,
        scratch_types=[
            pltpu.VMEM((n_chunks, _CHUNK), jnp.int32),
            pltpu.VMEM((n_chunks, _CHUNK), jnp.int32),
            pltpu.VMEM((n_chunks, _CHUNK), jnp.float32),
            pltpu.VMEM((n_pad // _N_SUBCORES,), jnp.float32),
            pltpu.VMEM((n_pad // _N_SUBCORES,), jnp.float32),
            pltpu.VMEM_SHARED((n_pad,), jnp.float32),
            pltpu.VMEM_SHARED((n_pad, half), jnp.float32),
            pltpu.SemaphoreType.DMA,
            pltpu.SemaphoreType.DMA,
            pltpu.SemaphoreType.DMA,
            pltpu.SemaphoreType.DMA,
            pltpu.SemaphoreType.DMA,
        ],
    )
    def k(s1_hbm, s2_hbm, h0_hbm, h1_hbm, dst_hbm, src_hbm, out_hbm,
          dstv, srcv, e_v, dn_v, rec_v, denom_sp, out_sp,
          dsem, gsem0, gsem1, ssem0, ssem1):
        core = lax.axis_index("core")
        t = lax.axis_index("subcore")
        base = t * rows_per_tile
        zeros16 = jnp.zeros((16,), jnp.float32)

        pltpu.sync_copy(dst_hbm.at[t], dstv)
        pltpu.sync_copy(src_hbm.at[t], srcv)

        @pl.loop(0, rows_per_tile // 16)
        def _(g):
            dn_v[pl.ds(g * 16, 16)] = zeros16

        pltpu.sync_copy(dn_v, denom_sp.at[pl.ds(base, rows_per_tile)])

        def phase1(s1_v, s2_v):
            pltpu.sync_copy(s1_hbm, s1_v)
            pltpu.sync_copy(s2_hbm, s2_v)
            plsc.subcore_barrier()

            @pl.loop(0, n_chunks)
            def _(c):
                for g in range(_CHUNK // 16):
                    d16 = dstv[c, pl.ds(g * 16, 16)]
                    sx16 = srcv[c, pl.ds(g * 16, 16)]
                    a = plsc.load_gather(s1_v, [d16])
                    b = plsc.load_gather(s2_v, [sx16])
                    x = a + b
                    x = jnp.maximum(x, x * 0.2)
                    x = jnp.minimum(jnp.maximum(x, -2.0), 2.0)
                    e_v[c, pl.ds(g * 16, 16)] = jnp.exp(x)
                pltpu.async_copy(e_v.at[c], denom_sp.at[dstv.at[c]], dsem, add=True)

                @pl.when(c >= 8)
                def _():
                    pltpu.make_async_copy(e_v.at[0], denom_sp.at[dstv.at[0]], dsem).wait()

            for _d in range(8):
                pltpu.make_async_copy(e_v.at[0], denom_sp.at[dstv.at[0]], dsem).wait()

        pl.run_scoped(phase1,
                      pltpu.VMEM((n_nodes,), jnp.float32),
                      pltpu.VMEM((n_nodes,), jnp.float32))

        def phase2(gbuf0, gbuf1, sbuf0, sbuf1):
            @pl.loop(0, _CHUNK)
            def _(j):
                for q in range(half // 16):
                    sbuf0[j, pl.ds(q * 16, 16)] = zeros16

            for ksub in range(n_sub):
                pltpu.sync_copy(sbuf0, out_sp.at[pl.ds(base + ksub * _CHUNK, _CHUNK)])

            plsc.subcore_barrier()

            bufs = ((gbuf0, gsem0, sbuf0, ssem0), (gbuf1, gsem1, sbuf1, ssem1))

            def p2(h_ref):
                pltpu.async_copy(h_ref.at[srcv.at[0]], gbuf0, gsem0)
                pltpu.async_copy(h_ref.at[srcv.at[1]], gbuf1, gsem1)

                @pl.loop(0, n_chunks, step=2)
                def _(c):
                    for b, (gbuf, gsem, sbuf, ssem) in enumerate(bufs):
                        cc = c + b
                        pltpu.make_async_copy(h_ref.at[srcv.at[cc]], gbuf, gsem).wait()

                        @pl.when(c >= 2)
                        def _():
                            pltpu.make_async_copy(sbuf, out_sp.at[dstv.at[cc]], ssem).wait()

                        for g in range(_CHUNK // 16):
                            e16 = e_v[cc, pl.ds(g * 16, 16)]
                            for jj in range(16):
                                j = g * 16 + jj
                                ej = e16[jj]
                                for q in range(half // 16):
                                    sbuf[j, pl.ds(q * 16, 16)] = gbuf[j, pl.ds(q * 16, 16)] * ej

                        @pl.when(cc + 2 < n_chunks)
                        def _():
                            pltpu.async_copy(h_ref.at[srcv.at[cc + 2]], gbuf, gsem)

                        pltpu.async_copy(sbuf, out_sp.at[dstv.at[cc]], ssem, add=True)

                for _b, (gbuf, gsem, sbuf, ssem) in enumerate(bufs):
                    pltpu.make_async_copy(sbuf, out_sp.at[dstv.at[0]], ssem).wait()

            @pl.when(core == 0)
            def _():
                p2(h0_hbm)

            @pl.when(core == 1)
            def _():
                p2(h1_hbm)

            plsc.subcore_barrier()

            pltpu.sync_copy(denom_sp.at[pl.ds(base, rows_per_tile)], dn_v)

            @pl.loop(0, rows_per_tile // 16)
            def _(g):
                d16 = dn_v[pl.ds(g * 16, 16)]
                rec_v[pl.ds(g * 16, 16)] = 1.0 / jnp.maximum(d16, 1e-20)

            for ksub in range(n_sub):
                off = base + ksub * _CHUNK
                pltpu.sync_copy(out_sp.at[pl.ds(off, _CHUNK)], gbuf0)

                for g in range(_CHUNK // 16):
                    r16 = rec_v[pl.ds(ksub * _CHUNK + g * 16, 16)]
                    for jj in range(16):
                        j = g * 16 + jj
                        rj = r16[jj]
                        for q in range(half // 16):
                            gbuf0[j, pl.ds(q * 16, 16)] = gbuf0[j, pl.ds(q * 16, 16)] * rj

                pltpu.sync_copy(gbuf0, out_hbm.at[core, pl.ds(off, _CHUNK)])

        pl.run_scoped(phase2,
                      pltpu.VMEM((_CHUNK, half), jnp.float32),
                      pltpu.VMEM((_CHUNK, half), jnp.float32),
                      pltpu.VMEM((_CHUNK, half), jnp.float32),
                      pltpu.VMEM((_CHUNK, half), jnp.float32))

    return k(s1, s2, h0, h1, dst3, src3)


def kernel(node_states, edges, kernel, kernel_attention):
    n_nodes, _ = node_states.shape
    n_edges = edges.shape[0]
    n_pad = ((n_nodes + 16 * _CHUNK - 1) // (16 * _CHUNK)) * (16 * _CHUNK)
    per_tile = n_edges // _N_SUBCORES
    n_chunks = per_tile // _CHUNK

    dst3 = edges[:, 0].reshape(_N_SUBCORES, n_chunks, _CHUNK)
    src3 = edges[:, 1].reshape(_N_SUBCORES, n_chunks, _CHUNK)
    a2 = kernel_attention.reshape(2, kernel.shape[1])

    h0, h1, s = _tc_transform(node_states, kernel, a2)
    out2 = _sc_gat(s[:, 0], s[:, 1], h0, h1, dst3, src3, n_nodes, n_pad)
    return jnp.concatenate([out2[0, :n_nodes], out2[1, :n_nodes]], axis=1)
